# baseline (speedup 1.0000x reference)
import functools

import jax
import jax.numpy as jnp
from jax import lax
from jax.experimental import pallas as pl
from jax.experimental.pallas import tpu as pltpu
from jax.experimental.pallas import tpu_sc as plsc

N = 10000
NP = 10240
E = 320000
U = 128
NC = 2
NS = 16
NW = NC * NS
EPT = 10240
E2 = EPT * NW
PADN = 10200
CH = 128
NCHUNK = EPT // CH
RPT = NP // NS
BN = 1024

_mesh = plsc.VectorSubcoreMesh(core_axis_name="c", subcore_axis_name="s")


def _tc_feats_body(x_ref, w_ref, as_ref, ad_ref, hs_ref, s_ref, d_ref):
    hsb = jnp.dot(x_ref[...], w_ref[0], preferred_element_type=jnp.float32)
    hs_ref[0] = hsb
    s_ref[...] = jnp.sum(hsb * as_ref[0, 0][None, :], axis=1)[None, None, :]
    d_ref[...] = jnp.sum(hsb * ad_ref[0, 0][None, :], axis=1)[None, None, :]


def _tc_feats(xp, W, a_s, a_d):
    H, K, _ = W.shape
    return pl.pallas_call(
        _tc_feats_body,
        grid=(H, NP // BN),
        in_specs=[
            pl.BlockSpec((BN, K), lambda h, i: (i, 0)),
            pl.BlockSpec((1, K, U), lambda h, i: (h, 0, 0)),
            pl.BlockSpec((1, 1, U), lambda h, i: (h, 0, 0)),
            pl.BlockSpec((1, 1, U), lambda h, i: (h, 0, 0)),
        ],
        out_specs=[
            pl.BlockSpec((1, BN, U), lambda h, i: (h, i, 0)),
            pl.BlockSpec((1, 1, BN), lambda h, i: (h, 0, i)),
            pl.BlockSpec((1, 1, BN), lambda h, i: (h, 0, i)),
        ],
        out_shape=[
            jax.ShapeDtypeStruct((H, NP, U), jnp.float32),
            jax.ShapeDtypeStruct((H, 1, NP), jnp.float32),
            jax.ShapeDtypeStruct((H, 1, NP), jnp.float32),
        ],
    )(xp, W, a_s.reshape(H, 1, U), a_d.reshape(H, 1, U))


def _sc_logits_body(H, src_hbm, dst_hbm, s_hbm, d_hbm, p_hbm, dp_hbm,
                    srcv, dstv, sv, dv, denv, pv):
    cid = lax.axis_index("c")
    sid = lax.axis_index("s")
    wid = cid * NS + sid
    base = wid * EPT
    pltpu.sync_copy(src_hbm.at[pl.ds(base, EPT)], srcv)
    pltpu.sync_copy(dst_hbm.at[pl.ds(base, EPT)], dstv)
    zero16 = jnp.zeros((16,), jnp.float32)
    for h in range(H):
        pltpu.sync_copy(s_hbm.at[pl.ds(h * NP, NP)], sv)
        pltpu.sync_copy(d_hbm.at[pl.ds(h * NP, NP)], dv)

        def zbody(i, _):
            denv[pl.ds(i * 16, 16)] = zero16
            return _
        lax.fori_loop(0, NP // 16, zbody, None)

        def ebody(i, _):
            ids = srcv[pl.ds(i * 16, 16)]
            idd = dstv[pl.ds(i * 16, 16)]
            svv = plsc.load_gather(sv, [ids])
            dvv = plsc.load_gather(dv, [idd])
            pre = svv + dvv
            act = jnp.where(pre > 0, pre, 0.2 * pre)
            pch = jnp.exp(act)
            pv[pl.ds(i * 16, 16)] = pch
            plsc.addupdate_scatter(denv, [idd], pch)
            return _
        lax.fori_loop(0, EPT // 16, ebody, None)

        pltpu.sync_copy(pv, p_hbm.at[pl.ds(h * E2 + base, EPT)])
        pltpu.sync_copy(denv, dp_hbm.at[pl.ds((h * NW + wid) * NP, NP)])


def _sc_logits(H, src, dst, s, d):
    f = pl.kernel(
        functools.partial(_sc_logits_body, H),
        out_type=[
            jax.ShapeDtypeStruct((H * E2,), jnp.float32),
            jax.ShapeDtypeStruct((H * NW * NP,), jnp.float32),
        ],
        mesh=_mesh,
        scratch_types=[
            pltpu.VMEM((EPT,), jnp.int32),
            pltpu.VMEM((EPT,), jnp.int32),
            pltpu.VMEM((NP,), jnp.float32),
            pltpu.VMEM((NP,), jnp.float32),
            pltpu.VMEM((NP,), jnp.float32),
            pltpu.VMEM((EPT,), jnp.float32),
        ],
        compiler_params=pltpu.CompilerParams(needs_layout_passes=False),
    )
    return f(src, dst, s, d)


def _tc_recip_body(dp_ref, r_ref):
    tot = jnp.sum(dp_ref[0], axis=0)
    r_ref[...] = (1.0 / (tot + 1e-9))[None, None, :]


def _tc_recip(H, dp):
    return pl.pallas_call(
        _tc_recip_body,
        grid=(H, NP // BN),
        in_specs=[pl.BlockSpec((1, NW, BN), lambda h, i: (h, 0, i))],
        out_specs=pl.BlockSpec((1, 1, BN), lambda h, i: (h, 0, i)),
        out_shape=jax.ShapeDtypeStruct((H, 1, NP), jnp.float32),
    )(dp)


def _sc_agg_body(H, src_hbm, dst_hbm, p_hbm, zz_hbm, hs_hbm, out_hbm,
                 pvt, srcv, dstv, srcx, dstx, rowsv, out_sh, gsem, ssem):
    cid = lax.axis_index("c")
    sid = lax.axis_index("s")
    wid = cid * NS + sid
    base = wid * EPT
    pltpu.sync_copy(src_hbm.at[pl.ds(base, EPT)], srcv)
    pltpu.sync_copy(dst_hbm.at[pl.ds(base, EPT)], dstv)

    def hbody(h, _carry):
        pltpu.sync_copy(zz_hbm.at[pl.ds(sid * RPT, RPT)],
                        out_sh.at[pl.ds(sid * RPT, RPT)])
        plsc.subcore_barrier()
        pltpu.sync_copy(p_hbm.at[pl.ds(h * E2 + base, EPT)], pvt)
        hoff = h * NP

        def cbody(c, _):
            cvecs = []
            for k in range(CH // 16):
                srcx[pl.ds(k * 16, 16)] = (
                    srcv[pl.ds(c * CH + k * 16, 16)] + hoff)
                dstx[pl.ds(k * 16, 16)] = dstv[pl.ds(c * CH + k * 16, 16)]
                cvecs.append(pvt[pl.ds(c * CH + k * 16, 16)])
            pltpu.async_copy(hs_hbm.at[srcx], rowsv, gsem).wait()

            def sbody(k5, _s):
                cvec = pvt[pl.ds(c * CH + k5 * 16, 16)]
                for rr in range(16):
                    r = k5 * 16 + rr
                    cb16 = jnp.full((16,), cvec[rr], jnp.float32)
                    for k in range(U // 16):
                        rowsv[r, pl.ds(k * 16, 16)] = (
                            rowsv[r, pl.ds(k * 16, 16)] * cb16)
                return _s
            lax.fori_loop(0, CH // 16, sbody, None)
            pltpu.async_copy(rowsv, out_sh.at[dstx], ssem, add=True).wait()
            return _
        lax.fori_loop(0, NCHUNK, cbody, None)
        plsc.subcore_barrier()
        pltpu.sync_copy(out_sh.at[pl.ds(sid * RPT, RPT)],
                        out_hbm.at[h, cid, pl.ds(sid * RPT, RPT)])
        plsc.subcore_barrier()
        return _carry
    lax.fori_loop(0, H, hbody, None)


def _sc_agg(H, srcf, dstf, p, hsflat, zz):
    f = pl.kernel(
        functools.partial(_sc_agg_body, H),
        out_type=jax.ShapeDtypeStruct((H, NC, NP, U), jnp.float32),
        mesh=_mesh,
        scratch_types=[
            pltpu.VMEM((EPT,), jnp.float32),
            pltpu.VMEM((EPT,), jnp.int32),
            pltpu.VMEM((EPT,), jnp.int32),
            pltpu.VMEM((CH,), jnp.int32),
            pltpu.VMEM((CH,), jnp.int32),
            pltpu.VMEM((CH, U), jnp.float32),
            pltpu.VMEM_SHARED((NP, U), jnp.float32),
            pltpu.SemaphoreType.DMA,
            pltpu.SemaphoreType.DMA,
        ],
        compiler_params=pltpu.CompilerParams(needs_layout_passes=False),
    )
    return f(srcf, dstf, p, zz, hsflat)


def _tc_merge_concat_body(H, has_res, part_ref, r_ref, x_ref, wres_ref,
                          o_ref):
    for h in range(H):
        rcol = r_ref[h][:, None]
        o_ref[:, h * U:(h + 1) * U] = (part_ref[h, 0] + part_ref[h, 1]) * rcol
    if has_res:
        o_ref[...] += jnp.dot(x_ref[...], wres_ref[...],
                              preferred_element_type=jnp.float32)
    else:
        o_ref[...] += x_ref[...]


def _tc_merge_concat(H, part, recip, xp, Wres):
    K = xp.shape[1]
    rr = recip.reshape(H, NP)
    if Wres is None:
        body = functools.partial(
            lambda H, p, r, x, o: _tc_merge_concat_body(H, False, p, r, x,
                                                        None, o), H)
        in_specs = [
            pl.BlockSpec((H, NC, BN, U), lambda i: (0, 0, i, 0)),
            pl.BlockSpec((H, BN), lambda i: (0, i)),
            pl.BlockSpec((BN, K), lambda i: (i, 0)),
        ]
        args = (part, rr, xp)
    else:
        body = functools.partial(_tc_merge_concat_body, H, True)
        in_specs = [
            pl.BlockSpec((H, NC, BN, U), lambda i: (0, 0, i, 0)),
            pl.BlockSpec((H, BN), lambda i: (0, i)),
            pl.BlockSpec((BN, K), lambda i: (i, 0)),
            pl.BlockSpec(Wres.shape, lambda i: (0, 0)),
        ]
        args = (part, rr, xp, Wres)
    return pl.pallas_call(
        body,
        grid=(NP // BN,),
        in_specs=in_specs,
        out_specs=pl.BlockSpec((BN, H * U), lambda i: (i, 0)),
        out_shape=jax.ShapeDtypeStruct((NP, H * U), jnp.float32),
    )(*args)


def _tc_merge_avg_body(H, part_ref, r_ref, x_ref, wres_ref, o_ref):
    acc = (part_ref[0, 0] + part_ref[0, 1]) * r_ref[0][:, None]
    for h in range(1, H):
        acc += (part_ref[h, 0] + part_ref[h, 1]) * r_ref[h][:, None]
    o_ref[...] = acc * (1.0 / H) + jnp.dot(
        x_ref[...], wres_ref[...], preferred_element_type=jnp.float32)


def _tc_merge_avg(H, part, recip, xp, Wres):
    K = xp.shape[1]
    rr = recip.reshape(H, NP)
    return pl.pallas_call(
        functools.partial(_tc_merge_avg_body, H),
        grid=(NP // BN,),
        in_specs=[
            pl.BlockSpec((H, NC, BN, U), lambda i: (0, 0, i, 0)),
            pl.BlockSpec((H, BN), lambda i: (0, i)),
            pl.BlockSpec((BN, K), lambda i: (i, 0)),
            pl.BlockSpec(Wres.shape, lambda i: (0, 0)),
        ],
        out_specs=pl.BlockSpec((BN, U), lambda i: (i, 0)),
        out_shape=jax.ShapeDtypeStruct((NP, U), jnp.float32),
    )(part, rr, xp, Wres)


def _gat_layer(xp, W, a_s, a_d, srcf, dstf, zz, merge, Wres):
    H = W.shape[0]
    hs, s, d = _tc_feats(xp, W, a_s, a_d)
    p, dp = _sc_logits(H, srcf, dstf, s.reshape(H * NP), d.reshape(H * NP))
    recip = _tc_recip(H, dp.reshape(H, NW, NP))
    part = _sc_agg(H, srcf, dstf, p, hs.reshape(H * NP, U), zz)
    if merge == 'concat':
        return _tc_merge_concat(H, part, recip, xp, Wres)
    return _tc_merge_avg(H, part, recip, xp, Wres)


def kernel(x, W1, a1s, a1d, Wres1, W2, a2s, a2d, W3, a3s, a3d, Wres3, edges):
    xp = jnp.pad(x, ((0, NP - N), (0, 0)))
    pad = jnp.full((NW, EPT - E // NW), PADN, jnp.int32)
    src2 = jnp.concatenate([edges[0].reshape(NW, E // NW), pad], axis=1)
    dst2 = jnp.concatenate([edges[1].reshape(NW, E // NW), pad], axis=1)
    srcf = src2.reshape(E2)
    dstf = dst2.reshape(E2)
    zz = jnp.zeros((NP, U), jnp.float32)
    x1 = _gat_layer(xp, W1, a1s, a1d, srcf, dstf, zz, 'concat', Wres1)
    x2 = _gat_layer(x1, W2, a2s, a2d, srcf, dstf, zz, 'concat', None)
    out = _gat_layer(x2, W3, a3s, a3d, srcf, dstf, zz, 'avg', Wres3)
    return out[:N]


# speedup vs baseline: 1.9987x; 1.9987x over previous
import functools

import jax
import jax.numpy as jnp
from jax import lax
from jax.experimental import pallas as pl
from jax.experimental.pallas import tpu as pltpu
from jax.experimental.pallas import tpu_sc as plsc

N = 10000
NP = 10240
E = 320000
U = 128
NC = 2
NS = 16
NW = NC * NS
EPT = 10240
E2 = EPT * NW
PADN = 10200
CH = 80
EPTB = 10000
NCHUNK = EPTB // CH
RPT = NP // NS
BN = 1024

_mesh = plsc.VectorSubcoreMesh(core_axis_name="c", subcore_axis_name="s")


def _tc_feats_body(x_ref, w_ref, as_ref, ad_ref, hs_ref, s_ref, d_ref):
    hsb = jnp.dot(x_ref[...], w_ref[0], preferred_element_type=jnp.float32)
    hs_ref[0] = hsb
    s_ref[...] = jnp.sum(hsb * as_ref[0, 0][None, :], axis=1)[None, None, :]
    d_ref[...] = jnp.sum(hsb * ad_ref[0, 0][None, :], axis=1)[None, None, :]


def _tc_feats(xp, W, a_s, a_d):
    H, K, _ = W.shape
    return pl.pallas_call(
        _tc_feats_body,
        grid=(H, NP // BN),
        in_specs=[
            pl.BlockSpec((BN, K), lambda h, i: (i, 0)),
            pl.BlockSpec((1, K, U), lambda h, i: (h, 0, 0)),
            pl.BlockSpec((1, 1, U), lambda h, i: (h, 0, 0)),
            pl.BlockSpec((1, 1, U), lambda h, i: (h, 0, 0)),
        ],
        out_specs=[
            pl.BlockSpec((1, BN, U), lambda h, i: (h, i, 0)),
            pl.BlockSpec((1, 1, BN), lambda h, i: (h, 0, i)),
            pl.BlockSpec((1, 1, BN), lambda h, i: (h, 0, i)),
        ],
        out_shape=[
            jax.ShapeDtypeStruct((H, NP, U), jnp.float32),
            jax.ShapeDtypeStruct((H, 1, NP), jnp.float32),
            jax.ShapeDtypeStruct((H, 1, NP), jnp.float32),
        ],
    )(xp, W, a_s.reshape(H, 1, U), a_d.reshape(H, 1, U))


def _sc_logits_body(H, src_hbm, dst_hbm, s_hbm, d_hbm, p_hbm, dp_hbm,
                    srcv, dstv, sv, dv, denv, pv):
    cid = lax.axis_index("c")
    sid = lax.axis_index("s")
    wid = cid * NS + sid
    base = wid * EPT
    pltpu.sync_copy(src_hbm.at[pl.ds(base, EPT)], srcv)
    pltpu.sync_copy(dst_hbm.at[pl.ds(base, EPT)], dstv)
    zero16 = jnp.zeros((16,), jnp.float32)
    for h in range(H):
        pltpu.sync_copy(s_hbm.at[pl.ds(h * NP, NP)], sv)
        pltpu.sync_copy(d_hbm.at[pl.ds(h * NP, NP)], dv)

        def zbody(i, _):
            denv[pl.ds(i * 16, 16)] = zero16
            return _
        lax.fori_loop(0, NP // 16, zbody, None)

        def ebody(i, _):
            ids = srcv[pl.ds(i * 16, 16)]
            idd = dstv[pl.ds(i * 16, 16)]
            svv = plsc.load_gather(sv, [ids])
            dvv = plsc.load_gather(dv, [idd])
            pre = svv + dvv
            act = jnp.where(pre > 0, pre, 0.2 * pre)
            pch = jnp.exp(act)
            pv[pl.ds(i * 16, 16)] = pch
            plsc.addupdate_scatter(denv, [idd], pch)
            return _
        lax.fori_loop(0, EPT // 16, ebody, None)

        pltpu.sync_copy(pv, p_hbm.at[pl.ds(h * E2 + base, EPT)])
        pltpu.sync_copy(denv, dp_hbm.at[pl.ds((h * NW + wid) * NP, NP)])


def _sc_logits(H, src, dst, s, d):
    f = pl.kernel(
        functools.partial(_sc_logits_body, H),
        out_type=[
            jax.ShapeDtypeStruct((H * E2,), jnp.float32),
            jax.ShapeDtypeStruct((H * NW * NP,), jnp.float32),
        ],
        mesh=_mesh,
        scratch_types=[
            pltpu.VMEM((EPT,), jnp.int32),
            pltpu.VMEM((EPT,), jnp.int32),
            pltpu.VMEM((NP,), jnp.float32),
            pltpu.VMEM((NP,), jnp.float32),
            pltpu.VMEM((NP,), jnp.float32),
            pltpu.VMEM((EPT,), jnp.float32),
        ],
        compiler_params=pltpu.CompilerParams(needs_layout_passes=False),
    )
    return f(src, dst, s, d)


def _tc_recip_body(dp_ref, r_ref):
    tot = jnp.sum(dp_ref[0], axis=0)
    r_ref[...] = (1.0 / (tot + 1e-9))[None, None, :]


def _tc_recip(H, dp):
    return pl.pallas_call(
        _tc_recip_body,
        grid=(H, NP // BN),
        in_specs=[pl.BlockSpec((1, NW, BN), lambda h, i: (h, 0, i))],
        out_specs=pl.BlockSpec((1, 1, BN), lambda h, i: (h, 0, i)),
        out_shape=jax.ShapeDtypeStruct((H, 1, NP), jnp.float32),
    )(dp)


def _sc_agg_body(H, *refs):
    (src_hbm, dst_hbm, p_hbm, zz_hbm) = refs[:4]
    hs_hbms = refs[4:4 + H]
    out_hbm = refs[4 + H]
    (pvt, srcx0, srcx1, dstx0, dstx1, rows0, rows1,
     out_sh, gsem0, gsem1, ssem) = refs[5 + H:]
    rows = (rows0, rows1)
    srcxs = (srcx0, srcx1)
    dstxs = (dstx0, dstx1)
    gsems = (gsem0, gsem1)
    cid = lax.axis_index("c")
    sid = lax.axis_index("s")
    wid = cid * NS + sid
    base = wid * EPTB

    def idx_load(c, b):
        pltpu.sync_copy(src_hbm.at[pl.ds(base + c * CH, CH)], srcxs[b])
        pltpu.sync_copy(dst_hbm.at[pl.ds(base + c * CH, CH)], dstxs[b])

    def wait_g(b):
        pltpu.make_async_copy(zz_hbm.at[pl.ds(0, CH)], rows[b],
                              gsems[b]).wait()

    def scale(c, b):
        rb = rows[b]

        def sbody(k5, _s):
            cvec = pvt[pl.ds(c * CH + k5 * 16, 16)]
            for rr in range(16):
                r = k5 * 16 + rr
                cb16 = jnp.full((16,), cvec[rr], jnp.float32)
                for k in range(U // 16):
                    rb[r, pl.ds(k * 16, 16)] = rb[r, pl.ds(k * 16, 16)] * cb16
            return _s
        lax.fori_loop(0, CH // 16, sbody, None)

    for h in range(H):
        hsh = hs_hbms[h]
        pltpu.sync_copy(zz_hbm.at[pl.ds(sid * RPT, RPT)],
                        out_sh.at[pl.ds(sid * RPT, RPT)])
        plsc.subcore_barrier()
        pltpu.sync_copy(p_hbm.at[pl.ds(h * E2 + wid * EPT, EPTB)], pvt)

        idx_load(0, 0)
        pltpu.async_copy(hsh.at[srcx0], rows[0], gsems[0])

        def cbody(j, _):
            c0 = j * 2
            idx_load(c0 + 1, 1)
            pltpu.async_copy(hsh.at[srcx1], rows[1], gsems[1])
            wait_g(0)
            scale(c0, 0)
            pltpu.async_copy(rows[0], out_sh.at[dstx0], ssem, add=True).wait()
            idx_load(c0 + 2, 0)
            pltpu.async_copy(hsh.at[srcx0], rows[0], gsems[0])
            wait_g(1)
            scale(c0 + 1, 1)
            pltpu.async_copy(rows[1], out_sh.at[dstx1], ssem, add=True).wait()
            return _
        lax.fori_loop(0, (NCHUNK - 1) // 2, cbody, None)
        wait_g(0)
        scale(NCHUNK - 1, 0)
        pltpu.async_copy(rows[0], out_sh.at[dstx0], ssem, add=True).wait()
        plsc.subcore_barrier()
        pltpu.sync_copy(out_sh.at[pl.ds(sid * RPT, RPT)],
                        out_hbm.at[h, cid, pl.ds(sid * RPT, RPT)])
        plsc.subcore_barrier()


def _sc_agg(H, srcu, dstu, p, hs_list, zz):
    f = pl.kernel(
        functools.partial(_sc_agg_body, H),
        out_type=jax.ShapeDtypeStruct((H, NC, NP, U), jnp.float32),
        mesh=_mesh,
        scratch_types=(
            [pltpu.VMEM((EPTB,), jnp.float32)]
            + [pltpu.VMEM((CH,), jnp.int32) for _ in range(4)]
            + [pltpu.VMEM((CH, U), jnp.float32) for _ in range(2)]
            + [pltpu.VMEM_SHARED((NP, U), jnp.float32)]
            + [pltpu.SemaphoreType.DMA for _ in range(3)]
        ),
        compiler_params=pltpu.CompilerParams(needs_layout_passes=False),
    )
    return f(srcu, dstu, p, zz, *hs_list)


def _tc_merge_concat_body(H, has_res, part_ref, r_ref, x_ref, wres_ref,
                          o_ref):
    for h in range(H):
        rcol = r_ref[h][:, None]
        o_ref[:, h * U:(h + 1) * U] = (part_ref[h, 0] + part_ref[h, 1]) * rcol
    if has_res:
        o_ref[...] += jnp.dot(x_ref[...], wres_ref[...],
                              preferred_element_type=jnp.float32)
    else:
        o_ref[...] += x_ref[...]


def _tc_merge_concat(H, part, recip, xp, Wres):
    K = xp.shape[1]
    rr = recip.reshape(H, NP)
    if Wres is None:
        body = functools.partial(
            lambda H, p, r, x, o: _tc_merge_concat_body(H, False, p, r, x,
                                                        None, o), H)
        in_specs = [
            pl.BlockSpec((H, NC, BN, U), lambda i: (0, 0, i, 0)),
            pl.BlockSpec((H, BN), lambda i: (0, i)),
            pl.BlockSpec((BN, K), lambda i: (i, 0)),
        ]
        args = (part, rr, xp)
    else:
        body = functools.partial(_tc_merge_concat_body, H, True)
        in_specs = [
            pl.BlockSpec((H, NC, BN, U), lambda i: (0, 0, i, 0)),
            pl.BlockSpec((H, BN), lambda i: (0, i)),
            pl.BlockSpec((BN, K), lambda i: (i, 0)),
            pl.BlockSpec(Wres.shape, lambda i: (0, 0)),
        ]
        args = (part, rr, xp, Wres)
    return pl.pallas_call(
        body,
        grid=(NP // BN,),
        in_specs=in_specs,
        out_specs=pl.BlockSpec((BN, H * U), lambda i: (i, 0)),
        out_shape=jax.ShapeDtypeStruct((NP, H * U), jnp.float32),
    )(*args)


def _tc_merge_avg_body(H, part_ref, r_ref, x_ref, wres_ref, o_ref):
    acc = (part_ref[0, 0] + part_ref[0, 1]) * r_ref[0][:, None]
    for h in range(1, H):
        acc += (part_ref[h, 0] + part_ref[h, 1]) * r_ref[h][:, None]
    o_ref[...] = acc * (1.0 / H) + jnp.dot(
        x_ref[...], wres_ref[...], preferred_element_type=jnp.float32)


def _tc_merge_avg(H, part, recip, xp, Wres):
    K = xp.shape[1]
    rr = recip.reshape(H, NP)
    return pl.pallas_call(
        functools.partial(_tc_merge_avg_body, H),
        grid=(NP // BN,),
        in_specs=[
            pl.BlockSpec((H, NC, BN, U), lambda i: (0, 0, i, 0)),
            pl.BlockSpec((H, BN), lambda i: (0, i)),
            pl.BlockSpec((BN, K), lambda i: (i, 0)),
            pl.BlockSpec(Wres.shape, lambda i: (0, 0)),
        ],
        out_specs=pl.BlockSpec((BN, U), lambda i: (i, 0)),
        out_shape=jax.ShapeDtypeStruct((NP, U), jnp.float32),
    )(part, rr, xp, Wres)


def _gat_layer(xp, W, a_s, a_d, srcf, dstf, srcu, dstu, zz, merge, Wres):
    H = W.shape[0]
    hs, s, d = _tc_feats(xp, W, a_s, a_d)
    p, dp = _sc_logits(H, srcf, dstf, s.reshape(H * NP), d.reshape(H * NP))
    recip = _tc_recip(H, dp.reshape(H, NW, NP))
    hs_list = [hs[h] for h in range(H)]
    part = _sc_agg(H, srcu, dstu, p, hs_list, zz)
    if merge == 'concat':
        return _tc_merge_concat(H, part, recip, xp, Wres)
    return _tc_merge_avg(H, part, recip, xp, Wres)


def kernel(x, W1, a1s, a1d, Wres1, W2, a2s, a2d, W3, a3s, a3d, Wres3, edges):
    xp = jnp.pad(x, ((0, NP - N), (0, 0)))
    pad = jnp.full((NW, EPT - E // NW), PADN, jnp.int32)
    src2 = jnp.concatenate([edges[0].reshape(NW, E // NW), pad], axis=1)
    dst2 = jnp.concatenate([edges[1].reshape(NW, E // NW), pad], axis=1)
    srcf = src2.reshape(E2)
    dstf = dst2.reshape(E2)
    srcu = edges[0]
    dstu = edges[1]
    zz = jnp.zeros((NP, U), jnp.float32)
    x1 = _gat_layer(xp, W1, a1s, a1d, srcf, dstf, srcu, dstu, zz,
                    'concat', Wres1)
    x2 = _gat_layer(x1, W2, a2s, a2d, srcf, dstf, srcu, dstu, zz,
                    'concat', None)
    out = _gat_layer(x2, W3, a3s, a3d, srcf, dstf, srcu, dstu, zz,
                     'avg', Wres3)
    return out[:N]


# speedup vs baseline: 2.4443x; 1.2229x over previous
import functools

import jax
import jax.numpy as jnp
from jax import lax
from jax.experimental import pallas as pl
from jax.experimental.pallas import tpu as pltpu
from jax.experimental.pallas import tpu_sc as plsc

N = 10000
NP = 10240
E = 320000
U = 128
NC = 2
NS = 16
NW = NC * NS
EPT = 10240
E2 = EPT * NW
PADN = 10200
CH = 80
EPTB = 10000
NCHUNK = EPTB // CH
RPT = NP // NS
BN = 1024

_mesh = plsc.VectorSubcoreMesh(core_axis_name="c", subcore_axis_name="s")


def _tc_feats_body(x_ref, w_ref, as_ref, ad_ref, hs_ref, s_ref, d_ref):
    hsb = jnp.dot(x_ref[...], w_ref[0], preferred_element_type=jnp.float32)
    hs_ref[0] = hsb
    s_ref[...] = jnp.sum(hsb * as_ref[0, 0][None, :], axis=1)[None, None, :]
    d_ref[...] = jnp.sum(hsb * ad_ref[0, 0][None, :], axis=1)[None, None, :]


def _tc_feats(xp, W, a_s, a_d):
    H, K, _ = W.shape
    return pl.pallas_call(
        _tc_feats_body,
        grid=(H, NP // BN),
        in_specs=[
            pl.BlockSpec((BN, K), lambda h, i: (i, 0)),
            pl.BlockSpec((1, K, U), lambda h, i: (h, 0, 0)),
            pl.BlockSpec((1, 1, U), lambda h, i: (h, 0, 0)),
            pl.BlockSpec((1, 1, U), lambda h, i: (h, 0, 0)),
        ],
        out_specs=[
            pl.BlockSpec((1, BN, U), lambda h, i: (h, i, 0)),
            pl.BlockSpec((1, 1, BN), lambda h, i: (h, 0, i)),
            pl.BlockSpec((1, 1, BN), lambda h, i: (h, 0, i)),
        ],
        out_shape=[
            jax.ShapeDtypeStruct((H, NP, U), jnp.float32),
            jax.ShapeDtypeStruct((H, 1, NP), jnp.float32),
            jax.ShapeDtypeStruct((H, 1, NP), jnp.float32),
        ],
    )(xp, W, a_s.reshape(H, 1, U), a_d.reshape(H, 1, U))


def _sc_logits_body(H, src_hbm, dst_hbm, s_hbm, d_hbm, p_hbm, dp_hbm,
                    srcv, dstv, sv, dv, denv, pv):
    cid = lax.axis_index("c")
    sid = lax.axis_index("s")
    wid = cid * NS + sid
    base = wid * EPT
    pltpu.sync_copy(src_hbm.at[pl.ds(base, EPT)], srcv)
    pltpu.sync_copy(dst_hbm.at[pl.ds(base, EPT)], dstv)
    zero16 = jnp.zeros((16,), jnp.float32)
    for h in range(H):
        pltpu.sync_copy(s_hbm.at[pl.ds(h * NP, NP)], sv)
        pltpu.sync_copy(d_hbm.at[pl.ds(h * NP, NP)], dv)

        def zbody(i, _):
            denv[pl.ds(i * 16, 16)] = zero16
            return _
        lax.fori_loop(0, NP // 16, zbody, None)

        def ebody(i, _):
            ids = srcv[pl.ds(i * 16, 16)]
            idd = dstv[pl.ds(i * 16, 16)]
            svv = plsc.load_gather(sv, [ids])
            dvv = plsc.load_gather(dv, [idd])
            pre = svv + dvv
            act = jnp.where(pre > 0, pre, 0.2 * pre)
            pch = jnp.exp(act)
            pv[pl.ds(i * 16, 16)] = pch
            plsc.addupdate_scatter(denv, [idd], pch)
            return _
        lax.fori_loop(0, EPT // 16, ebody, None)

        pltpu.sync_copy(pv, p_hbm.at[pl.ds(h * E2 + base, EPT)])
        pltpu.sync_copy(denv, dp_hbm.at[pl.ds((h * NW + wid) * NP, NP)])


def _sc_logits(H, src, dst, s, d):
    f = pl.kernel(
        functools.partial(_sc_logits_body, H),
        out_type=[
            jax.ShapeDtypeStruct((H * E2,), jnp.float32),
            jax.ShapeDtypeStruct((H * NW * NP,), jnp.float32),
        ],
        mesh=_mesh,
        scratch_types=[
            pltpu.VMEM((EPT,), jnp.int32),
            pltpu.VMEM((EPT,), jnp.int32),
            pltpu.VMEM((NP,), jnp.float32),
            pltpu.VMEM((NP,), jnp.float32),
            pltpu.VMEM((NP,), jnp.float32),
            pltpu.VMEM((EPT,), jnp.float32),
        ],
        compiler_params=pltpu.CompilerParams(needs_layout_passes=False),
    )
    return f(src, dst, s, d)


def _tc_recip_body(dp_ref, r_ref):
    tot = jnp.sum(dp_ref[0], axis=0)
    r_ref[...] = (1.0 / (tot + 1e-9))[None, None, :]


def _tc_recip(H, dp):
    return pl.pallas_call(
        _tc_recip_body,
        grid=(H, NP // BN),
        in_specs=[pl.BlockSpec((1, NW, BN), lambda h, i: (h, 0, i))],
        out_specs=pl.BlockSpec((1, 1, BN), lambda h, i: (h, 0, i)),
        out_shape=jax.ShapeDtypeStruct((H, 1, NP), jnp.float32),
    )(dp)


def _sc_agg_body(H, *refs):
    (src_hbm, dst_hbm, p_hbm, zz_hbm) = refs[:4]
    hs_hbms = refs[4:4 + H]
    out_hbm = refs[4 + H]
    (pvt, srcx0, srcx1, dstx0, dstx1, rows0, rows1,
     out_sh, gsem0, gsem1, ssem0, ssem1, isem0, isem1) = refs[5 + H:]
    rows = (rows0, rows1)
    srcxs = (srcx0, srcx1)
    dstxs = (dstx0, dstx1)
    gsems = (gsem0, gsem1)
    ssems = (ssem0, ssem1)
    isems = (isem0, isem1)
    cid = lax.axis_index("c")
    sid = lax.axis_index("s")
    wid = cid * NS + sid
    base = wid * EPTB

    def idx_load(c, b):
        pltpu.async_copy(src_hbm.at[pl.ds(base + c * CH, CH)], srcxs[b],
                         isems[b])
        pltpu.async_copy(dst_hbm.at[pl.ds(base + c * CH, CH)], dstxs[b],
                         isems[b])

    def wait_i(b):
        pltpu.make_async_copy(src_hbm.at[pl.ds(0, CH)], srcxs[b],
                              isems[b]).wait()
        pltpu.make_async_copy(dst_hbm.at[pl.ds(0, CH)], dstxs[b],
                              isems[b]).wait()

    def wait_g(b):
        pltpu.make_async_copy(zz_hbm.at[pl.ds(0, CH)], rows[b],
                              gsems[b]).wait()

    def wait_s(b):
        pltpu.make_async_copy(zz_hbm.at[pl.ds(0, CH)], rows[b],
                              ssems[b]).wait()

    def scale(c, b):
        rb = rows[b]

        def sbody(k5, _s):
            cvec = pvt[pl.ds(c * CH + k5 * 16, 16)]
            for rr in range(16):
                r = k5 * 16 + rr
                cb16 = jnp.full((16,), cvec[rr], jnp.float32)
                for k in range(U // 16):
                    rb[r, pl.ds(k * 16, 16)] = rb[r, pl.ds(k * 16, 16)] * cb16
            return _s
        lax.fori_loop(0, CH // 16, sbody, None)

    for h in range(H):
        hsh = hs_hbms[h]
        pltpu.sync_copy(zz_hbm.at[pl.ds(sid * RPT, RPT)],
                        out_sh.at[pl.ds(sid * RPT, RPT)])
        plsc.subcore_barrier()
        pltpu.sync_copy(p_hbm.at[pl.ds(h * E2 + wid * EPT, EPTB)], pvt)

        idx_load(0, 0)
        idx_load(1, 1)
        wait_i(0)
        pltpu.async_copy(hsh.at[srcxs[0]], rows[0], gsems[0])
        wait_i(1)
        pltpu.async_copy(hsh.at[srcxs[1]], rows[1], gsems[1])

        def cbody(j, _):
            c0 = j * 2
            wait_g(0)
            scale(c0, 0)
            pltpu.async_copy(rows[0], out_sh.at[dstxs[0]], ssems[0],
                             add=True)
            wait_g(1)
            scale(c0 + 1, 1)
            pltpu.async_copy(rows[1], out_sh.at[dstxs[1]], ssems[1],
                             add=True)
            wait_s(0)
            idx_load(c0 + 2, 0)
            wait_i(0)
            pltpu.async_copy(hsh.at[srcxs[0]], rows[0], gsems[0])
            wait_s(1)
            cn = c0 + 3
            cs = jnp.where(cn < NCHUNK, cn, 0)
            idx_load(cs, 1)
            wait_i(1)
            pltpu.async_copy(hsh.at[srcxs[1]], rows[1], gsems[1])
            return _
        lax.fori_loop(0, (NCHUNK - 1) // 2, cbody, None)
        wait_g(0)
        scale(NCHUNK - 1, 0)
        pltpu.async_copy(rows[0], out_sh.at[dstxs[0]], ssems[0],
                         add=True).wait()
        wait_g(1)
        plsc.subcore_barrier()
        pltpu.sync_copy(out_sh.at[pl.ds(sid * RPT, RPT)],
                        out_hbm.at[h, cid, pl.ds(sid * RPT, RPT)])
        plsc.subcore_barrier()


def _sc_agg(H, srcu, dstu, p, hs_list, zz):
    f = pl.kernel(
        functools.partial(_sc_agg_body, H),
        out_type=jax.ShapeDtypeStruct((H, NC, NP, U), jnp.float32),
        mesh=_mesh,
        scratch_types=(
            [pltpu.VMEM((EPTB,), jnp.float32)]
            + [pltpu.VMEM((CH,), jnp.int32) for _ in range(4)]
            + [pltpu.VMEM((CH, U), jnp.float32) for _ in range(2)]
            + [pltpu.VMEM_SHARED((NP, U), jnp.float32)]
            + [pltpu.SemaphoreType.DMA for _ in range(6)]
        ),
        compiler_params=pltpu.CompilerParams(needs_layout_passes=False),
    )
    return f(srcu, dstu, p, zz, *hs_list)


def _tc_merge_concat_body(H, has_res, part_ref, r_ref, x_ref, wres_ref,
                          o_ref):
    for h in range(H):
        rcol = r_ref[h][:, None]
        o_ref[:, h * U:(h + 1) * U] = (part_ref[h, 0] + part_ref[h, 1]) * rcol
    if has_res:
        o_ref[...] += jnp.dot(x_ref[...], wres_ref[...],
                              preferred_element_type=jnp.float32)
    else:
        o_ref[...] += x_ref[...]


def _tc_merge_concat(H, part, recip, xp, Wres):
    K = xp.shape[1]
    rr = recip.reshape(H, NP)
    if Wres is None:
        body = functools.partial(
            lambda H, p, r, x, o: _tc_merge_concat_body(H, False, p, r, x,
                                                        None, o), H)
        in_specs = [
            pl.BlockSpec((H, NC, BN, U), lambda i: (0, 0, i, 0)),
            pl.BlockSpec((H, BN), lambda i: (0, i)),
            pl.BlockSpec((BN, K), lambda i: (i, 0)),
        ]
        args = (part, rr, xp)
    else:
        body = functools.partial(_tc_merge_concat_body, H, True)
        in_specs = [
            pl.BlockSpec((H, NC, BN, U), lambda i: (0, 0, i, 0)),
            pl.BlockSpec((H, BN), lambda i: (0, i)),
            pl.BlockSpec((BN, K), lambda i: (i, 0)),
            pl.BlockSpec(Wres.shape, lambda i: (0, 0)),
        ]
        args = (part, rr, xp, Wres)
    return pl.pallas_call(
        body,
        grid=(NP // BN,),
        in_specs=in_specs,
        out_specs=pl.BlockSpec((BN, H * U), lambda i: (i, 0)),
        out_shape=jax.ShapeDtypeStruct((NP, H * U), jnp.float32),
    )(*args)


def _tc_merge_avg_body(H, part_ref, r_ref, x_ref, wres_ref, o_ref):
    acc = (part_ref[0, 0] + part_ref[0, 1]) * r_ref[0][:, None]
    for h in range(1, H):
        acc += (part_ref[h, 0] + part_ref[h, 1]) * r_ref[h][:, None]
    o_ref[...] = acc * (1.0 / H) + jnp.dot(
        x_ref[...], wres_ref[...], preferred_element_type=jnp.float32)


def _tc_merge_avg(H, part, recip, xp, Wres):
    K = xp.shape[1]
    rr = recip.reshape(H, NP)
    return pl.pallas_call(
        functools.partial(_tc_merge_avg_body, H),
        grid=(NP // BN,),
        in_specs=[
            pl.BlockSpec((H, NC, BN, U), lambda i: (0, 0, i, 0)),
            pl.BlockSpec((H, BN), lambda i: (0, i)),
            pl.BlockSpec((BN, K), lambda i: (i, 0)),
            pl.BlockSpec(Wres.shape, lambda i: (0, 0)),
        ],
        out_specs=pl.BlockSpec((BN, U), lambda i: (i, 0)),
        out_shape=jax.ShapeDtypeStruct((NP, U), jnp.float32),
    )(part, rr, xp, Wres)


def _gat_layer(xp, W, a_s, a_d, srcf, dstf, srcu, dstu, zz, merge, Wres):
    H = W.shape[0]
    hs, s, d = _tc_feats(xp, W, a_s, a_d)
    p, dp = _sc_logits(H, srcf, dstf, s.reshape(H * NP), d.reshape(H * NP))
    recip = _tc_recip(H, dp.reshape(H, NW, NP))
    hs_list = [hs[h] for h in range(H)]
    part = _sc_agg(H, srcu, dstu, p, hs_list, zz)
    if merge == 'concat':
        return _tc_merge_concat(H, part, recip, xp, Wres)
    return _tc_merge_avg(H, part, recip, xp, Wres)


def kernel(x, W1, a1s, a1d, Wres1, W2, a2s, a2d, W3, a3s, a3d, Wres3, edges):
    xp = jnp.pad(x, ((0, NP - N), (0, 0)))
    pad = jnp.full((NW, EPT - E // NW), PADN, jnp.int32)
    src2 = jnp.concatenate([edges[0].reshape(NW, E // NW), pad], axis=1)
    dst2 = jnp.concatenate([edges[1].reshape(NW, E // NW), pad], axis=1)
    srcf = src2.reshape(E2)
    dstf = dst2.reshape(E2)
    srcu = edges[0]
    dstu = edges[1]
    zz = jnp.zeros((NP, U), jnp.float32)
    x1 = _gat_layer(xp, W1, a1s, a1d, srcf, dstf, srcu, dstu, zz,
                    'concat', Wres1)
    x2 = _gat_layer(x1, W2, a2s, a2d, srcf, dstf, srcu, dstu, zz,
                    'concat', None)
    out = _gat_layer(x2, W3, a3s, a3d, srcf, dstf, srcu, dstu, zz,
                     'avg', Wres3)
    return out[:N]
